# baseline (speedup 1.0000x reference)
import functools

import jax
import jax.numpy as jnp
from jax import lax
from jax.experimental import pallas as pl
from jax.experimental.pallas import tpu as pltpu
from jax.experimental.pallas import tpu_sc as plsc

NC = 2
NS = 16
NW = NC * NS

CHUNK = 128


def _sc_gather(tab, idx, n_rows, edim):
    idx_w = n_rows // NW
    half = idx_w // 2
    n_chunk = half // CHUNK

    mesh = plsc.VectorSubcoreMesh(core_axis_name="c", subcore_axis_name="s")

    @functools.partial(
        pl.kernel,
        mesh=mesh,
        compiler_params=pltpu.CompilerParams(use_tc_tiling_on_sc=False),
        out_type=jax.ShapeDtypeStruct((n_rows, edim), jnp.float32),
        scratch_types=[
            pltpu.VMEM((idx_w,), jnp.int32),
            pltpu.VMEM((2, half, edim), jnp.float32),
            pltpu.SemaphoreType.DMA,
            pltpu.SemaphoreType.DMA,
            pltpu.SemaphoreType.DMA,
            pltpu.SemaphoreType.DMA,
        ],
    )
    def k(tab_hbm, idx_hbm, out_hbm, idx_v, buf, g0, g1, o0, o1):
        wid = lax.axis_index("s") * NC + lax.axis_index("c")
        base = wid * idx_w
        gsems = (g0, g1)
        osems = (o0, o1)

        pltpu.sync_copy(idx_hbm.at[pl.ds(base, idx_w)], idx_v)

        def fire(b):
            def body(c, carry):
                off = pl.multiple_of(c * CHUNK, CHUNK)
                pltpu.async_copy(
                    tab_hbm.at[idx_v.at[pl.ds(b * half + off, CHUNK)]],
                    buf.at[b, pl.ds(off, CHUNK)],
                    gsems[b],
                )
                return carry

            lax.fori_loop(0, n_chunk, body, 0)

        def drain_gathers(b):
            pltpu.make_async_copy(
                tab_hbm.at[pl.ds(0, half)], buf.at[b], gsems[b]
            ).wait()

        def flush(b):
            pltpu.async_copy(
                buf.at[b], out_hbm.at[pl.ds(base + b * half, half)], osems[b]
            )

        def drain_flush(b):
            pltpu.make_async_copy(
                buf.at[b], out_hbm.at[pl.ds(base + b * half, half)], osems[b]
            ).wait()

        fire(0)
        fire(1)
        drain_gathers(0)
        flush(0)
        drain_gathers(1)
        flush(1)
        drain_flush(0)
        drain_flush(1)

    return k(tab, idx)


def _tc_transpose_table(tab_t, nt, vocab, edim):
    n_grp = 128 // edim
    super_w = n_grp * 128
    nsuper = -(-vocab // super_w)
    ksuper = 7
    gcols = -(-nsuper // ksuper)

    def body(in_ref, out_ref):
        x = in_ref[0]
        eye = jnp.eye(128, 128, dtype=jnp.float32)
        dn = (((0,), (0,)), ((), ()))
        for i in range(ksuper):
            x8 = jnp.concatenate(
                [x[:, i * super_w + s * 128:i * super_w + (s + 1) * 128]
                 for s in range(n_grp)],
                axis=0,
            )
            z = jax.lax.dot_general(
                x8, eye, dn, preferred_element_type=jnp.float32
            )
            out_ref[0, i * 128:(i + 1) * 128, :] = z

    return pl.pallas_call(
        body,
        grid=(nt, gcols),
        in_specs=[
            pl.BlockSpec((1, edim, ksuper * super_w), lambda j, b: (j, 0, b))
        ],
        out_specs=pl.BlockSpec(
            (1, ksuper * 128, 128), lambda j, b: (j, b, 0)
        ),
        out_shape=jax.ShapeDtypeStruct(
            (nt, gcols * ksuper * 128, 128), jnp.float32
        ),
    )(tab_t)


def _tc_mlp(qemb, pemb, qin, pin, a_q, a_p, d_q, d_p, b1, w2, b2, w3, b3):
    bsz, ke = qemb.shape
    kd = qin.shape[1]
    h1 = a_q.shape[1]
    h2 = w2.shape[1]
    bb = 2048
    grid = (bsz // bb,)

    def body(qe, pe, qi, pi, aq, ap, dq, dp, b1r, w2r, b2r, w3r, b3r, out):
        h = (
            jnp.dot(qe[...], aq[...], preferred_element_type=jnp.float32)
            + jnp.dot(pe[...], ap[...], preferred_element_type=jnp.float32)
            + jnp.dot(qi[...], dq[...], preferred_element_type=jnp.float32)
            + jnp.dot(pi[...], dp[...], preferred_element_type=jnp.float32)
            + b1r[...]
        )
        h = jnp.tanh(h)
        h = jnp.tanh(
            jnp.dot(h, w2r[...], preferred_element_type=jnp.float32) + b2r[...]
        )
        o = jnp.dot(h, w3r[...], preferred_element_type=jnp.float32) + b3r[...]
        out[...] = 1.0 / (1.0 + jnp.exp(-o))

    full = lambda shape: pl.BlockSpec(shape, lambda i: (0, 0))
    return pl.pallas_call(
        body,
        grid=grid,
        in_specs=[
            pl.BlockSpec((bb, ke), lambda i: (i, 0)),
            pl.BlockSpec((bb, ke), lambda i: (i, 0)),
            pl.BlockSpec((bb, kd), lambda i: (i, 0)),
            pl.BlockSpec((bb, kd), lambda i: (i, 0)),
            full(a_q.shape),
            full(a_p.shape),
            full(d_q.shape),
            full(d_p.shape),
            full((1, h1)),
            full(w2.shape),
            full((1, h2)),
            full(w3.shape),
            full((1, 1)),
        ],
        out_specs=pl.BlockSpec((bb, 1), lambda i: (i, 0)),
        out_shape=jax.ShapeDtypeStruct((bsz, 1), jnp.float32),
    )(qemb, pemb, qin, pin, a_q, a_p, d_q, d_p, b1, w2, b2, w3, b3)


def kernel(que_inputs, pro_inputs, que_tables, pro_tables, W1, b1, W2, b2, W3, b3):
    bsz, qdim = que_inputs.shape
    pdim = pro_inputs.shape[1]
    nq, vocab, edim = que_tables.shape
    npro = pro_tables.shape[0]

    qt = _tc_transpose_table(jnp.transpose(que_tables, (0, 2, 1)), nq, vocab, edim)
    pt = _tc_transpose_table(jnp.transpose(pro_tables, (0, 2, 1)), npro, vocab, edim)
    vocab_pad = qt.shape[1] * qt.shape[2] // edim
    qflat = qt.reshape(nq * vocab_pad, edim)
    pflat = pt.reshape(npro * vocab_pad, edim)

    n_grp = 128 // edim
    super_w = n_grp * 128
    offs_q = (jnp.arange(nq, dtype=jnp.int32) * vocab_pad)[None, :]
    offs_p = (jnp.arange(npro, dtype=jnp.int32) * vocab_pad)[None, :]

    def _perm(v):
        return ((v // super_w) * super_w
                + (v % 128) * n_grp
                + (v // 128) % n_grp)

    qv = que_inputs[:, :nq].astype(jnp.int32)
    pv = pro_inputs[:, :npro].astype(jnp.int32)
    qidx = (_perm(qv) + offs_q).reshape(-1)
    pidx = (_perm(pv) + offs_p).reshape(-1)

    qrows = _sc_gather(qflat, qidx, bsz * nq, edim)
    prows = _sc_gather(pflat, pidx, bsz * npro, edim)
    qemb = qrows.reshape(bsz, nq * edim)
    pemb = prows.reshape(bsz, npro * edim)

    ne_q, ne_p = nq * edim, npro * edim
    a_q = W1[:ne_q]
    d_q = jnp.zeros((qdim, W1.shape[1]), jnp.float32).at[nq:].set(
        W1[ne_q : ne_q + (qdim - nq)]
    )
    a_p = W1[ne_q + (qdim - nq) : ne_q + (qdim - nq) + ne_p]
    d_p = jnp.zeros((pdim, W1.shape[1]), jnp.float32).at[npro:].set(
        W1[ne_q + (qdim - nq) + ne_p :]
    )

    return _tc_mlp(
        qemb, pemb, que_inputs, pro_inputs,
        a_q, a_p, d_q, d_p,
        b1.reshape(1, -1), W2, b2.reshape(1, -1), W3, b3.reshape(1, -1),
    )


# speedup vs baseline: 1.6893x; 1.6893x over previous
import functools

import jax
import jax.numpy as jnp
from jax import lax
from jax.experimental import pallas as pl
from jax.experimental.pallas import tpu as pltpu
from jax.experimental.pallas import tpu_sc as plsc

NC = 2
NS = 16
NW = NC * NS

CHUNK = 128


def _sc_gather(tab, idx, n_rows, edim):
    idx_w = n_rows // NW
    half = idx_w // 2
    n_chunk = half // CHUNK

    mesh = plsc.VectorSubcoreMesh(core_axis_name="c", subcore_axis_name="s")

    @functools.partial(
        pl.kernel,
        mesh=mesh,
        compiler_params=pltpu.CompilerParams(use_tc_tiling_on_sc=False),
        out_type=jax.ShapeDtypeStruct((n_rows, edim), jnp.float32),
        scratch_types=[
            pltpu.VMEM((idx_w,), jnp.int32),
            pltpu.VMEM((2, half, edim), jnp.float32),
            pltpu.SemaphoreType.DMA,
            pltpu.SemaphoreType.DMA,
            pltpu.SemaphoreType.DMA,
            pltpu.SemaphoreType.DMA,
        ],
    )
    def k(tab_hbm, idx_hbm, out_hbm, idx_v, buf, g0, g1, o0, o1):
        wid = lax.axis_index("s") * NC + lax.axis_index("c")
        base = wid * idx_w
        gsems = (g0, g1)
        osems = (o0, o1)

        pltpu.sync_copy(idx_hbm.at[pl.ds(base, idx_w)], idx_v)

        def fire(b):
            def body(c, carry):
                off = pl.multiple_of(c * CHUNK, CHUNK)
                pltpu.async_copy(
                    tab_hbm.at[idx_v.at[pl.ds(b * half + off, CHUNK)]],
                    buf.at[b, pl.ds(off, CHUNK)],
                    gsems[b],
                )
                return carry

            lax.fori_loop(0, n_chunk, body, 0)

        def drain_gathers(b):
            pltpu.make_async_copy(
                tab_hbm.at[pl.ds(0, half)], buf.at[b], gsems[b]
            ).wait()

        def flush(b):
            pltpu.async_copy(
                buf.at[b], out_hbm.at[pl.ds(base + b * half, half)], osems[b]
            )

        def drain_flush(b):
            pltpu.make_async_copy(
                buf.at[b], out_hbm.at[pl.ds(base + b * half, half)], osems[b]
            ).wait()

        fire(0)
        fire(1)
        drain_gathers(0)
        flush(0)
        drain_gathers(1)
        flush(1)
        drain_flush(0)
        drain_flush(1)

    return k(tab, idx)


def _tc_transpose_table(tab_t, nt, vocab, edim):
    n_grp = 128 // edim
    super_w = n_grp * 128
    nsuper = -(-vocab // super_w)
    ksuper = 49
    gcols = -(-nsuper // ksuper)

    def body(in_ref, out_ref):
        x = in_ref[0]
        eye = jnp.eye(128, 128, dtype=jnp.float32)
        dn = (((0,), (0,)), ((), ()))
        for i in range(ksuper):
            x8 = jnp.concatenate(
                [x[:, i * super_w + s * 128:i * super_w + (s + 1) * 128]
                 for s in range(n_grp)],
                axis=0,
            )
            z = jax.lax.dot_general(
                x8, eye, dn, preferred_element_type=jnp.float32
            )
            out_ref[0, i * 128:(i + 1) * 128, :] = z

    return pl.pallas_call(
        body,
        grid=(nt, gcols),
        in_specs=[
            pl.BlockSpec((1, edim, ksuper * super_w), lambda j, b: (j, 0, b))
        ],
        out_specs=pl.BlockSpec(
            (1, ksuper * 128, 128), lambda j, b: (j, b, 0)
        ),
        out_shape=jax.ShapeDtypeStruct(
            (nt, gcols * ksuper * 128, 128), jnp.float32
        ),
    )(tab_t)


def _tc_mlp(qemb, pemb, qin, pin, a_q, a_p, d_q, d_p, b1, w2, b2, w3, b3):
    bsz, ke = qemb.shape
    kd = qin.shape[1]
    h1 = a_q.shape[1]
    h2 = w2.shape[1]
    bb = 2048
    grid = (bsz // bb,)

    def body(qe, pe, qi, pi, aq, ap, dq, dp, b1r, w2r, b2r, w3r, b3r, out):
        h = (
            jnp.dot(qe[...], aq[...], preferred_element_type=jnp.float32)
            + jnp.dot(pe[...], ap[...], preferred_element_type=jnp.float32)
            + jnp.dot(qi[...], dq[...], preferred_element_type=jnp.float32)
            + jnp.dot(pi[...], dp[...], preferred_element_type=jnp.float32)
            + b1r[...]
        )
        h = jnp.tanh(h)
        h = jnp.tanh(
            jnp.dot(h, w2r[...], preferred_element_type=jnp.float32) + b2r[...]
        )
        o = jnp.dot(h, w3r[...], preferred_element_type=jnp.float32) + b3r[...]
        out[...] = 1.0 / (1.0 + jnp.exp(-o))

    full = lambda shape: pl.BlockSpec(shape, lambda i: (0, 0))
    return pl.pallas_call(
        body,
        grid=grid,
        in_specs=[
            pl.BlockSpec((bb, ke), lambda i: (i, 0)),
            pl.BlockSpec((bb, ke), lambda i: (i, 0)),
            pl.BlockSpec((bb, kd), lambda i: (i, 0)),
            pl.BlockSpec((bb, kd), lambda i: (i, 0)),
            full(a_q.shape),
            full(a_p.shape),
            full(d_q.shape),
            full(d_p.shape),
            full((1, h1)),
            full(w2.shape),
            full((1, h2)),
            full(w3.shape),
            full((1, 1)),
        ],
        out_specs=pl.BlockSpec((bb, 1), lambda i: (i, 0)),
        out_shape=jax.ShapeDtypeStruct((bsz, 1), jnp.float32),
    )(qemb, pemb, qin, pin, a_q, a_p, d_q, d_p, b1, w2, b2, w3, b3)


def kernel(que_inputs, pro_inputs, que_tables, pro_tables, W1, b1, W2, b2, W3, b3):
    bsz, qdim = que_inputs.shape
    pdim = pro_inputs.shape[1]
    nq, vocab, edim = que_tables.shape
    npro = pro_tables.shape[0]

    qt = _tc_transpose_table(jnp.transpose(que_tables, (0, 2, 1)), nq, vocab, edim)
    pt = _tc_transpose_table(jnp.transpose(pro_tables, (0, 2, 1)), npro, vocab, edim)
    vocab_pad = qt.shape[1] * qt.shape[2] // edim
    qflat = qt.reshape(nq * vocab_pad, edim)
    pflat = pt.reshape(npro * vocab_pad, edim)

    n_grp = 128 // edim
    super_w = n_grp * 128
    offs_q = (jnp.arange(nq, dtype=jnp.int32) * vocab_pad)[None, :]
    offs_p = (jnp.arange(npro, dtype=jnp.int32) * vocab_pad)[None, :]

    def _perm(v):
        return ((v // super_w) * super_w
                + (v % 128) * n_grp
                + (v // 128) % n_grp)

    qv = que_inputs[:, :nq].astype(jnp.int32)
    pv = pro_inputs[:, :npro].astype(jnp.int32)
    qidx = (_perm(qv) + offs_q).reshape(-1)
    pidx = (_perm(pv) + offs_p).reshape(-1)

    qrows = _sc_gather(qflat, qidx, bsz * nq, edim)
    prows = _sc_gather(pflat, pidx, bsz * npro, edim)
    qemb = qrows.reshape(bsz, nq * edim)
    pemb = prows.reshape(bsz, npro * edim)

    ne_q, ne_p = nq * edim, npro * edim
    a_q = W1[:ne_q]
    d_q = jnp.zeros((qdim, W1.shape[1]), jnp.float32).at[nq:].set(
        W1[ne_q : ne_q + (qdim - nq)]
    )
    a_p = W1[ne_q + (qdim - nq) : ne_q + (qdim - nq) + ne_p]
    d_p = jnp.zeros((pdim, W1.shape[1]), jnp.float32).at[npro:].set(
        W1[ne_q + (qdim - nq) + ne_p :]
    )

    return _tc_mlp(
        qemb, pemb, que_inputs, pro_inputs,
        a_q, a_p, d_q, d_p,
        b1.reshape(1, -1), W2, b2.reshape(1, -1), W3, b3.reshape(1, -1),
    )


# speedup vs baseline: 1.6920x; 1.0016x over previous
import functools

import jax
import jax.numpy as jnp
from jax import lax
from jax.experimental import pallas as pl
from jax.experimental.pallas import tpu as pltpu
from jax.experimental.pallas import tpu_sc as plsc

NC = 2
NS = 16
NW = NC * NS

CHUNK = 128


def _sc_gather(tab, idx, n_rows, edim):
    idx_w = n_rows // NW
    half = idx_w // 2
    n_chunk = half // CHUNK

    mesh = plsc.VectorSubcoreMesh(core_axis_name="c", subcore_axis_name="s")

    @functools.partial(
        pl.kernel,
        mesh=mesh,
        compiler_params=pltpu.CompilerParams(use_tc_tiling_on_sc=False),
        out_type=jax.ShapeDtypeStruct((n_rows, edim), jnp.float32),
        scratch_types=[
            pltpu.VMEM((idx_w,), jnp.int32),
            pltpu.VMEM((2, half, edim), jnp.float32),
            pltpu.SemaphoreType.DMA,
            pltpu.SemaphoreType.DMA,
            pltpu.SemaphoreType.DMA,
            pltpu.SemaphoreType.DMA,
        ],
    )
    def k(tab_hbm, idx_hbm, out_hbm, idx_v, buf, g0, g1, o0, o1):
        wid = lax.axis_index("s") * NC + lax.axis_index("c")
        base = wid * idx_w
        gsems = (g0, g1)
        osems = (o0, o1)

        pltpu.sync_copy(idx_hbm.at[pl.ds(base, idx_w)], idx_v)

        def fire(b):
            def body(c, carry):
                off = pl.multiple_of(c * CHUNK, CHUNK)
                pltpu.async_copy(
                    tab_hbm.at[idx_v.at[pl.ds(b * half + off, CHUNK)]],
                    buf.at[b, pl.ds(off, CHUNK)],
                    gsems[b],
                )
                return carry

            lax.fori_loop(0, n_chunk, body, 0)

        def drain_gathers(b):
            pltpu.make_async_copy(
                tab_hbm.at[pl.ds(0, half)], buf.at[b], gsems[b]
            ).wait()

        def flush(b):
            pltpu.async_copy(
                buf.at[b], out_hbm.at[pl.ds(base + b * half, half)], osems[b]
            )

        def drain_flush(b):
            pltpu.make_async_copy(
                buf.at[b], out_hbm.at[pl.ds(base + b * half, half)], osems[b]
            ).wait()

        fire(0)
        fire(1)
        drain_gathers(0)
        flush(0)
        drain_gathers(1)
        flush(1)
        drain_flush(0)
        drain_flush(1)

    return k(tab, idx)


def _tc_transpose_table(tab_t, nt, vocab, edim):
    n_grp = 128 // edim
    super_w = n_grp * 128
    nsuper = -(-vocab // super_w)
    ksuper = 49
    gcols = -(-nsuper // ksuper)

    def body(in_ref, out_ref):
        x = in_ref[0]
        eye = jnp.eye(128, 128, dtype=jnp.float32)
        dn = (((0,), (0,)), ((), ()))
        for i in range(ksuper):
            x8 = jnp.concatenate(
                [x[:, i * super_w + s * 128:i * super_w + (s + 1) * 128]
                 for s in range(n_grp)],
                axis=0,
            )
            z = jax.lax.dot_general(
                x8, eye, dn, preferred_element_type=jnp.float32
            )
            out_ref[0, i * 128:(i + 1) * 128, :] = z

    return pl.pallas_call(
        body,
        grid=(nt, gcols),
        in_specs=[
            pl.BlockSpec((1, edim, ksuper * super_w), lambda j, b: (j, 0, b))
        ],
        out_specs=pl.BlockSpec(
            (1, ksuper * 128, 128), lambda j, b: (j, b, 0)
        ),
        out_shape=jax.ShapeDtypeStruct(
            (nt, gcols * ksuper * 128, 128), jnp.float32
        ),
    )(tab_t)


def _tc_mlp(qemb, pemb, db, a_q, a_p, w2, b2, w3, b3):
    bsz, ke = qemb.shape
    h1 = db.shape[1]
    h2 = w2.shape[1]
    bb = 2048
    grid = (bsz // bb,)

    def body(qe, pe, dbr, aq, ap, w2r, b2r, w3r, b3r, out):
        h = (
            jnp.dot(qe[...], aq[...], preferred_element_type=jnp.float32)
            + jnp.dot(pe[...], ap[...], preferred_element_type=jnp.float32)
            + dbr[...]
        )
        h = jnp.tanh(h)
        h = jnp.tanh(
            jnp.dot(h, w2r[...], preferred_element_type=jnp.float32) + b2r[...]
        )
        o = jnp.dot(h, w3r[...], preferred_element_type=jnp.float32) + b3r[...]
        out[...] = 1.0 / (1.0 + jnp.exp(-o))

    full = lambda shape: pl.BlockSpec(shape, lambda i: (0, 0))
    return pl.pallas_call(
        body,
        grid=grid,
        in_specs=[
            pl.BlockSpec((bb, ke), lambda i: (i, 0)),
            pl.BlockSpec((bb, ke), lambda i: (i, 0)),
            pl.BlockSpec((bb, h1), lambda i: (i, 0)),
            full(a_q.shape),
            full(a_p.shape),
            full(w2.shape),
            full((1, h2)),
            full(w3.shape),
            full((1, 1)),
        ],
        out_specs=pl.BlockSpec((bb, 1), lambda i: (i, 0)),
        out_shape=jax.ShapeDtypeStruct((bsz, 1), jnp.float32),
    )(qemb, pemb, db, a_q, a_p, w2, b2, w3, b3)


def kernel(que_inputs, pro_inputs, que_tables, pro_tables, W1, b1, W2, b2, W3, b3):
    bsz, qdim = que_inputs.shape
    pdim = pro_inputs.shape[1]
    nq, vocab, edim = que_tables.shape
    npro = pro_tables.shape[0]

    qt = _tc_transpose_table(jnp.transpose(que_tables, (0, 2, 1)), nq, vocab, edim)
    pt = _tc_transpose_table(jnp.transpose(pro_tables, (0, 2, 1)), npro, vocab, edim)
    vocab_pad = qt.shape[1] * qt.shape[2] // edim
    qflat = qt.reshape(nq * vocab_pad, edim)
    pflat = pt.reshape(npro * vocab_pad, edim)

    n_grp = 128 // edim
    super_w = n_grp * 128
    offs_q = (jnp.arange(nq, dtype=jnp.int32) * vocab_pad)[None, :]
    offs_p = (jnp.arange(npro, dtype=jnp.int32) * vocab_pad)[None, :]

    def _perm(v):
        return ((v // super_w) * super_w
                + (v % 128) * n_grp
                + (v // 128) % n_grp)

    qv = que_inputs[:, :nq].astype(jnp.int32)
    pv = pro_inputs[:, :npro].astype(jnp.int32)
    qidx = (_perm(qv) + offs_q).reshape(-1)
    pidx = (_perm(pv) + offs_p).reshape(-1)

    qrows = _sc_gather(qflat, qidx, bsz * nq, edim)
    prows = _sc_gather(pflat, pidx, bsz * npro, edim)
    qemb = qrows.reshape(bsz, nq * edim)
    pemb = prows.reshape(bsz, npro * edim)

    ne_q, ne_p = nq * edim, npro * edim
    a_q = W1[:ne_q]
    d_q = W1[ne_q : ne_q + (qdim - nq)]
    a_p = W1[ne_q + (qdim - nq) : ne_q + (qdim - nq) + ne_p]
    d_p = W1[ne_q + (qdim - nq) + ne_p :]
    db = que_inputs[:, nq:] @ d_q + pro_inputs[:, npro:] @ d_p + b1[None, :]

    return _tc_mlp(
        qemb, pemb, db,
        a_q, a_p, W2, b2.reshape(1, -1), W3, b3.reshape(1, -1),
    )
